# baseline (speedup 1.0000x reference)
import math

import jax
import jax.numpy as jnp
from jax import lax
from jax.experimental import pallas as pl
from jax.experimental.pallas import tpu as pltpu
from jax.experimental.pallas import tpu_sc as plsc

NUM_USER = 1000000
NUM_ITEM = 1000000
F = 64
B = 4096
REG = 0.01

NC = 2
NS = 16
NW = NC * NS
BPW = B // NW

_INV_LN2 = 1.0 / math.log(2.0)


def _sc_gather3(u, i, j, embed_user, embed_item):
    mesh = plsc.VectorSubcoreMesh(core_axis_name="c", subcore_axis_name="s")
    rows = jax.ShapeDtypeStruct((B, F), jnp.float32)

    @pl.kernel(
        out_type=[rows, rows, rows],
        mesh=mesh,
        compiler_params=pltpu.CompilerParams(use_tc_tiling_on_sc=False),
        scratch_types=[
            pltpu.VMEM((BPW,), jnp.int32),
            pltpu.VMEM((BPW,), jnp.int32),
            pltpu.VMEM((BPW,), jnp.int32),
            pltpu.VMEM((BPW, F), jnp.float32),
            pltpu.VMEM((BPW, F), jnp.float32),
            pltpu.VMEM((BPW, F), jnp.float32),
            pltpu.SemaphoreType.DMA,
            pltpu.SemaphoreType.DMA,
            pltpu.SemaphoreType.DMA,
        ],
    )
    def gather_kernel(u_hbm, i_hbm, j_hbm, user_hbm, item_hbm,
                      eu_hbm, ei_hbm, ej_hbm,
                      idx_u, idx_i, idx_j, rows_u, rows_i, rows_j,
                      sem_u, sem_i, sem_j):
        wid = lax.axis_index("s") * NC + lax.axis_index("c")
        sl = pl.ds(wid * BPW, BPW)
        pltpu.sync_copy(u_hbm.at[sl], idx_u)
        pltpu.sync_copy(i_hbm.at[sl], idx_i)
        pltpu.sync_copy(j_hbm.at[sl], idx_j)
        cu = pltpu.async_copy(user_hbm.at[idx_u], rows_u, sem_u)
        ci = pltpu.async_copy(item_hbm.at[idx_i], rows_i, sem_i)
        cj = pltpu.async_copy(item_hbm.at[idx_j], rows_j, sem_j)
        cu.wait()
        pltpu.sync_copy(rows_u, eu_hbm.at[sl])
        ci.wait()
        pltpu.sync_copy(rows_i, ei_hbm.at[sl])
        cj.wait()
        pltpu.sync_copy(rows_j, ej_hbm.at[sl])

    return gather_kernel(u, i, j, embed_user, embed_item)


def _tc_body(eu_ref, ei_ref, ej_ref, yui_ref, yuj_ref, loss_ref):
    eu = eu_ref[...]
    ei = ei_ref[...]
    ej = ej_ref[...]
    s_i = jnp.sum(ei, axis=0, keepdims=True)
    s_j = jnp.sum(ej, axis=0, keepdims=True)
    y_ui = jnp.sum(eu * s_i, axis=1)
    y_uj = jnp.sum(eu * s_j, axis=1)
    yui_ref[...] = y_ui
    yuj_ref[...] = y_uj
    reg = REG * (jnp.sum(eu * eu) + jnp.sum(ei * ei) + jnp.sum(ej * ej))
    d = y_ui - y_uj
    ls = jnp.minimum(d, 0.0) - jnp.log(1.0 + jnp.exp(-jnp.abs(d)))
    loss_ref[0, 0] = reg - jnp.sum(ls) * _INV_LN2


def _tc_finish(eu, ei, ej):
    return pl.pallas_call(
        _tc_body,
        out_shape=(
            jax.ShapeDtypeStruct((B,), jnp.float32),
            jax.ShapeDtypeStruct((B,), jnp.float32),
            jax.ShapeDtypeStruct((1, 1), jnp.float32),
        ),
        out_specs=(
            pl.BlockSpec(memory_space=pltpu.VMEM),
            pl.BlockSpec(memory_space=pltpu.VMEM),
            pl.BlockSpec(memory_space=pltpu.SMEM),
        ),
    )(eu, ei, ej)


def kernel(u, i, j, embed_user, embed_item):
    eu, ei, ej = _sc_gather3(u, i, j, embed_user, embed_item)
    y_ui, y_uj, loss2d = _tc_finish(eu, ei, ej)
    return y_ui, y_uj, loss2d[0, 0]
